# baseline (speedup 1.0000x reference)
import dataclasses

import numpy as np

import jax
import jax.numpy as jnp
from jax import lax
from jax.experimental import pallas as pl
from jax.experimental.pallas import tpu as pltpu
from jax.experimental.pallas import tpu_sc as plsc

N = 10000
E = 320000
D = 128

NC = 2
NS = 16
CW = 32
NCHUNK = 640
EPS = NCHUNK * CW
E_PAD = NS * EPS
GRP = 8
NGRP = NCHUNK // GRP
LANES = 16

HALF = 5120
ACC_ROWS = 5248
ZPS = ACC_ROWS // NS
FPS = HALF // NS
DUMP = HALF
TROWS = 5120
NPAD = NC * HALF

_PERM = np.zeros(D, dtype=np.int32)
for _k in range(4):
    for _j in range(16):
        _PERM[32 * _k + 2 * _j] = 32 * _k + _j
        _PERM[32 * _k + 2 * _j + 1] = 32 * _k + 16 + _j
PERM = _PERM


def _bcast_lane(vec, lane):
    idx = jnp.full((LANES, 1), lane, jnp.int32)
    dnums = lax.GatherDimensionNumbers(
        offset_dims=(), collapsed_slice_dims=(0,), start_index_map=(0,))
    return lax.gather(vec, idx, dnums, (1,),
                      mode=lax.GatherScatterMode.PROMISE_IN_BOUNDS)


def _mm1_body(x_ref, w_ref, o_ref):
    o_ref[...] = jnp.dot(x_ref[...], w_ref[...],
                         preferred_element_type=jnp.float32
                         ).astype(jnp.bfloat16)


def _mm2_body(p_ref, b_ref, w_ref, o_ref):
    h = p_ref[:N, :] + b_ref[...]
    h = jnp.maximum(h, 0.0)
    o_ref[...] = jnp.dot(h, w_ref[...], preferred_element_type=jnp.float32
                         ).astype(jnp.bfloat16)


def _final_body(p_ref, b_ref, o_ref):
    o_ref[...] = p_ref[:N, :] + b_ref[...]


def _tc_mm1(x, w):
    return pl.pallas_call(
        _mm1_body,
        out_shape=jax.ShapeDtypeStruct((N, D), jnp.bfloat16),
    )(x, w)


def _tc_mm2(p, b, w):
    return pl.pallas_call(
        _mm2_body,
        out_shape=jax.ShapeDtypeStruct((N, D), jnp.bfloat16),
    )(p, b, w)


def _tc_final(p, b):
    return pl.pallas_call(
        _final_body,
        out_shape=jax.ShapeDtypeStruct((N, D), jnp.float32),
    )(p, b)


def _pack_table(sup_bf):
    padded = jnp.pad(sup_bf, ((0, 2 * TROWS - N), (0, 0)))
    return lax.bitcast_convert_type(
        padded.reshape(TROWS, D, 2), jnp.int32)


def _sc_agg_body(table_hbm, srcp_hbm, dst_hbm, w_hbm, par_hbm, out_hbm,
                 srcp_v, dst_v, w_v, par_v, rows_v, sc_v,
                 table_sh, acc_sh, sg0, sg1, ss0, ss1, sem_i):
    cid = lax.axis_index("c")
    sid = lax.axis_index("s")
    sem_g = (sg0, sg1)
    sem_s = (ss0, ss1)
    dstbase = cid * HALF
    cols = [jnp.arange(LANES, dtype=jnp.int32) + 16 * k for k in range(4)]

    for b in range(2):
        @pl.loop(0, CW)
        def _zero_rows(r):
            for k in range(D // LANES):
                sc_v[b, r, pl.ds(k * LANES, LANES)] = jnp.zeros(
                    (LANES,), jnp.float32)

    zbase = sid * ZPS
    for i in range(ZPS // CW):
        pltpu.sync_copy(sc_v.at[0], acc_sh.at[pl.ds(zbase + i * CW, CW)])
    pltpu.sync_copy(sc_v.at[0].at[pl.ds(0, ZPS % CW)],
                    acc_sh.at[pl.ds(zbase + (ZPS // CW) * CW, ZPS % CW)])

    pltpu.sync_copy(table_hbm.at[pl.ds(sid * FPS, FPS)],
                    table_sh.at[pl.ds(sid * FPS, FPS)])

    pltpu.sync_copy(srcp_hbm.at[sid].at[pl.ds(0, GRP)], srcp_v.at[0])
    pltpu.sync_copy(dst_hbm.at[sid].at[pl.ds(0, GRP)], dst_v.at[0])
    pltpu.sync_copy(w_hbm.at[sid].at[pl.ds(0, GRP)], w_v.at[0])
    pltpu.sync_copy(par_hbm.at[sid].at[pl.ds(0, GRP)], par_v.at[0])

    def _localize_dst(buf):
        for r in range(GRP):
            for q in range(CW // LANES):
                sl = pl.ds(q * LANES, LANES)
                v0 = dst_v[buf, r, sl]
                v = v0 - dstbase
                ok = (v >= 0) & (v < HALF)
                dst_v[buf, r, sl] = jnp.where(ok, v, DUMP + (v0 & 127))

    _localize_dst(0)

    plsc.subcore_barrier()

    pltpu.async_copy(sc_v.at[1], acc_sh.at[dst_v.at[0, 0]], sem_s[1],
                     add=True)
    pltpu.async_copy(table_sh.at[srcp_v.at[0, 0]], rows_v.at[0], sem_g[0])

    @pl.loop(0, NCHUNK, step=2)
    def _pair(t):
        for b in range(2):
            jj = t + b
            gsel = (jj // GRP) % 2
            lrow = jj % GRP

            pltpu.make_async_copy(
                sc_v.at[1 - b], acc_sh.at[dst_v.at[0, 0]],
                sem_s[1 - b]).wait()

            @pl.when(lrow == 0)
            def _stage():
                g2 = jnp.minimum(jj // GRP + 1, NGRP - 1)
                off = g2 * GRP
                tgt = 1 - gsel
                pltpu.async_copy(srcp_hbm.at[sid].at[pl.ds(off, GRP)],
                                 srcp_v.at[tgt], sem_i)
                pltpu.async_copy(dst_hbm.at[sid].at[pl.ds(off, GRP)],
                                 dst_v.at[tgt], sem_i)
                pltpu.async_copy(w_hbm.at[sid].at[pl.ds(off, GRP)],
                                 w_v.at[tgt], sem_i)
                pltpu.async_copy(par_hbm.at[sid].at[pl.ds(off, GRP)],
                                 par_v.at[tgt], sem_i)

            @pl.when(lrow == GRP - 1)
            def _wait_stage():
                for _ in range(4):
                    pltpu.make_async_copy(
                        srcp_hbm.at[0].at[pl.ds(0, GRP)], srcp_v.at[0],
                        sem_i).wait()
                _localize_dst(1 - gsel)

            nj = jnp.minimum(jj + 1, NCHUNK - 1)
            ngsel = (nj // GRP) % 2
            nrow = nj % GRP
            pltpu.async_copy(table_sh.at[srcp_v.at[ngsel, nrow]],
                             rows_v.at[1 - b], sem_g[1 - b])

            pltpu.make_async_copy(
                table_sh.at[srcp_v.at[0, 0]], rows_v.at[b],
                sem_g[b]).wait()

            for g in range(CW // LANES):
                wv = w_v[gsel, lrow, pl.ds(g * LANES, LANES)]
                pv = par_v[gsel, lrow, pl.ds(g * LANES, LANES)]
                for e in range(LANES):
                    edge = g * LANES + e
                    wb = _bcast_lane(wv, e)
                    pb64 = _bcast_lane(pv, e) * 64
                    rowc = jnp.full((LANES,), edge, jnp.int32)
                    for k in range(4):
                        col = pb64 + cols[k]
                        x32 = plsc.load_gather(rows_v.at[b], [rowc, col])
                        ab = plsc.bitcast(x32, jnp.bfloat16)
                        lo, hi = plsc.unpack(
                            ab, format=plsc.PackFormat.INTERLEAVED)
                        sc_v[b, edge, pl.ds(32 * k, LANES)] = lo * wb
                        sc_v[b, edge, pl.ds(32 * k + 16, LANES)] = hi * wb

            pltpu.async_copy(sc_v.at[b],
                             acc_sh.at[dst_v.at[gsel, lrow]], sem_s[b],
                             add=True)

    pltpu.make_async_copy(
        sc_v.at[1], acc_sh.at[dst_v.at[0, 0]], sem_s[1]).wait()
    pltpu.make_async_copy(
        table_sh.at[srcp_v.at[0, 0]], rows_v.at[0], sem_g[0]).wait()

    plsc.subcore_barrier()

    pltpu.sync_copy(acc_sh.at[pl.ds(sid * FPS, FPS)],
                    out_hbm.at[pl.ds(dstbase + sid * FPS, FPS)])


@jax.jit
def _sc_aggregate(table, srcp3d, dst3d, w3d, par3d):
    mesh = plsc.VectorSubcoreMesh(core_axis_name="c", subcore_axis_name="s")
    cp = pltpu.CompilerParams()
    if "needs_layout_passes" in pltpu.CompilerParams.__dataclass_fields__:
        cp = dataclasses.replace(cp, needs_layout_passes=False)
    f = pl.kernel(
        _sc_agg_body,
        out_type=jax.ShapeDtypeStruct((NPAD, D), jnp.float32),
        mesh=mesh,
        compiler_params=cp,
        scratch_types=[
            pltpu.VMEM((2, GRP, CW), jnp.int32),
            pltpu.VMEM((2, GRP, CW), jnp.int32),
            pltpu.VMEM((2, GRP, CW), jnp.float32),
            pltpu.VMEM((2, GRP, CW), jnp.int32),
            pltpu.VMEM((2, CW, D), jnp.int32),
            pltpu.VMEM((2, CW, D), jnp.float32),
            pltpu.VMEM_SHARED((TROWS, D), jnp.int32),
            pltpu.VMEM_SHARED((ACC_ROWS, D), jnp.float32),
            pltpu.SemaphoreType.DMA,
            pltpu.SemaphoreType.DMA,
            pltpu.SemaphoreType.DMA,
            pltpu.SemaphoreType.DMA,
            pltpu.SemaphoreType.DMA,
        ],
    )
    return f(table, srcp3d, dst3d, w3d, par3d)


def kernel(x, edge_index, edge_weight, W1, b1, W2, b2):
    pad = E_PAD - E
    src = jnp.pad(edge_index[0], (0, pad))
    srcp3d = (src >> 1).reshape(NS, NCHUNK, CW)
    par3d = (src & 1).reshape(NS, NCHUNK, CW)
    dst3d = jnp.pad(edge_index[1], (0, pad)).reshape(NS, NCHUNK, CW)
    w3d = jnp.pad(edge_weight, (0, pad)).reshape(NS, NCHUNK, CW)
    b1r = b1.reshape(1, D)
    b2r = b2.reshape(1, D)
    W1p = W1[:, PERM]
    W2p = W2[:, PERM]

    t1 = _pack_table(_tc_mm1(x, W1p))
    p1 = _sc_aggregate(t1, srcp3d, dst3d, w3d, par3d)
    t2 = _pack_table(_tc_mm2(p1, b1r, W2p))
    p2 = _sc_aggregate(t2, srcp3d, dst3d, w3d, par3d)
    return _tc_final(p2, b2r)


# speedup vs baseline: 3.2923x; 3.2923x over previous
import jax
import jax.numpy as jnp
from jax import lax
from jax.experimental import pallas as pl
from jax.experimental.pallas import tpu as pltpu
from jax.experimental.pallas import tpu_sc as plsc

N = 10000
E = 320000
D = 128

NC = 2
NS = 16
NW = NC * NS
CW = 64
NCHUNK = 160
EPW = NCHUNK * CW
E_PAD = NW * EPW
GRP = 16
NGRP = NCHUNK // GRP
DEPTH = 4
NPAD = 10240
ROWS_PER_SUB = NPAD // NS
LANES = 16


def _bcast_lane(vec, lane):
    idx = jnp.full((LANES, 1), lane, jnp.int32)
    dnums = lax.GatherDimensionNumbers(
        offset_dims=(), collapsed_slice_dims=(0,), start_index_map=(0,))
    return lax.gather(vec, idx, dnums, (1,),
                      mode=lax.GatherScatterMode.PROMISE_IN_BOUNDS)


def _mm1_body(x_ref, w_ref, o_ref):
    o_ref[...] = jnp.dot(x_ref[...], w_ref[...],
                         preferred_element_type=jnp.float32)


def _mm2_body(p_ref, b_ref, w_ref, o_ref):
    h = p_ref[0, :N, :] + p_ref[1, :N, :] + b_ref[...]
    h = jnp.maximum(h, 0.0)
    o_ref[...] = jnp.dot(h, w_ref[...], preferred_element_type=jnp.float32)


def _final_body(p_ref, b_ref, o_ref):
    o_ref[...] = p_ref[0, :N, :] + p_ref[1, :N, :] + b_ref[...]


def _tc_mm1(x, w):
    return pl.pallas_call(
        _mm1_body,
        out_shape=jax.ShapeDtypeStruct((N, D), jnp.float32),
    )(x, w)


def _tc_mm2(p, b, w):
    return pl.pallas_call(
        _mm2_body,
        out_shape=jax.ShapeDtypeStruct((N, D), jnp.float32),
    )(p, b, w)


def _tc_final(p, b):
    return pl.pallas_call(
        _final_body,
        out_shape=jax.ShapeDtypeStruct((N, D), jnp.float32),
    )(p, b)


def _sc_agg_body(sup_hbm, src_hbm, dst_hbm, w_hbm, out_hbm,
                 src_v, dst_v, w_v, rows_v, acc_sh,
                 sg0, sg1, sg2, sg3, ss0, ss1, ss2, ss3, sem_i):
    cid = lax.axis_index("c")
    sid = lax.axis_index("s")
    wid = cid * NS + sid
    sem_g = (sg0, sg1, sg2, sg3)
    sem_s = (ss0, ss1, ss2, ss3)

    for b in range(DEPTH):
        @pl.loop(0, CW)
        def _zero_rows(r):
            for k in range(D // LANES):
                rows_v[b, r, pl.ds(k * LANES, LANES)] = jnp.zeros(
                    (LANES,), jnp.float32)

    base_row = sid * ROWS_PER_SUB
    for i in range(ROWS_PER_SUB // CW):
        pltpu.sync_copy(rows_v.at[0],
                        acc_sh.at[pl.ds(base_row + i * CW, CW)])

    pltpu.sync_copy(src_hbm.at[wid].at[pl.ds(0, GRP)], src_v.at[0])
    pltpu.sync_copy(dst_hbm.at[wid].at[pl.ds(0, GRP)], dst_v.at[0])
    pltpu.sync_copy(w_hbm.at[wid].at[pl.ds(0, GRP)], w_v.at[0])

    plsc.subcore_barrier()

    pltpu.async_copy(rows_v.at[DEPTH - 1], acc_sh.at[dst_v.at[0, 0]],
                     sem_s[DEPTH - 1], add=True)
    for b in range(DEPTH - 1):
        pltpu.async_copy(sup_hbm.at[src_v.at[0, b]], rows_v.at[b],
                         sem_g[b])

    @pl.loop(0, NCHUNK, step=DEPTH)
    def _quad(t):
        for b in range(DEPTH):
            jj = t + b
            gsel = (jj // GRP) % 2
            lrow = jj % GRP
            fb = (b + DEPTH - 1) % DEPTH

            pltpu.make_async_copy(
                rows_v.at[fb], acc_sh.at[dst_v.at[0, 0]], sem_s[fb]).wait()

            @pl.when(lrow == 0)
            def _stage():
                g2 = jnp.minimum(jj // GRP + 1, NGRP - 1)
                off = g2 * GRP
                tgt = 1 - gsel
                pltpu.async_copy(src_hbm.at[wid].at[pl.ds(off, GRP)],
                                 src_v.at[tgt], sem_i)
                pltpu.async_copy(dst_hbm.at[wid].at[pl.ds(off, GRP)],
                                 dst_v.at[tgt], sem_i)
                pltpu.async_copy(w_hbm.at[wid].at[pl.ds(off, GRP)],
                                 w_v.at[tgt], sem_i)

            @pl.when(lrow == GRP - DEPTH + 1)
            def _wait_stage():
                for _ in range(3):
                    pltpu.make_async_copy(
                        src_hbm.at[0].at[pl.ds(0, GRP)], src_v.at[0],
                        sem_i).wait()

            nj = jnp.minimum(jj + DEPTH - 1, NCHUNK - 1)
            ngsel = (nj // GRP) % 2
            nrow = nj % GRP
            pltpu.async_copy(sup_hbm.at[src_v.at[ngsel, nrow]],
                             rows_v.at[fb], sem_g[fb])

            pltpu.make_async_copy(
                sup_hbm.at[src_v.at[0, 0]], rows_v.at[b], sem_g[b]).wait()

            for g in range(CW // LANES):
                wv = w_v[gsel, lrow, pl.ds(g * LANES, LANES)]
                for e in range(LANES):
                    row = g * LANES + e
                    wb = _bcast_lane(wv, e)
                    for k in range(D // LANES):
                        sl = pl.ds(k * LANES, LANES)
                        rows_v[b, row, sl] = rows_v[b, row, sl] * wb

            pltpu.async_copy(rows_v.at[b],
                             acc_sh.at[dst_v.at[gsel, lrow]], sem_s[b],
                             add=True)

    pltpu.make_async_copy(
        rows_v.at[DEPTH - 1], acc_sh.at[dst_v.at[0, 0]],
        sem_s[DEPTH - 1]).wait()
    for b in range(DEPTH - 1):
        pltpu.make_async_copy(
            sup_hbm.at[src_v.at[0, 0]], rows_v.at[b], sem_g[b]).wait()

    plsc.subcore_barrier()

    pltpu.sync_copy(acc_sh.at[pl.ds(base_row, ROWS_PER_SUB)],
                    out_hbm.at[cid].at[pl.ds(base_row, ROWS_PER_SUB)])


@jax.jit
def _sc_aggregate(sup, src3d, dst3d, w3d):
    mesh = plsc.VectorSubcoreMesh(core_axis_name="c", subcore_axis_name="s")
    f = pl.kernel(
        _sc_agg_body,
        out_type=jax.ShapeDtypeStruct((NC, NPAD, D), jnp.float32),
        mesh=mesh,
        scratch_types=[
            pltpu.VMEM((2, GRP, CW), jnp.int32),
            pltpu.VMEM((2, GRP, CW), jnp.int32),
            pltpu.VMEM((2, GRP, CW), jnp.float32),
            pltpu.VMEM((DEPTH, CW, D), jnp.float32),
            pltpu.VMEM_SHARED((NPAD, D), jnp.float32),
            pltpu.SemaphoreType.DMA,
            pltpu.SemaphoreType.DMA,
            pltpu.SemaphoreType.DMA,
            pltpu.SemaphoreType.DMA,
            pltpu.SemaphoreType.DMA,
            pltpu.SemaphoreType.DMA,
            pltpu.SemaphoreType.DMA,
            pltpu.SemaphoreType.DMA,
            pltpu.SemaphoreType.DMA,
        ],
    )
    return f(sup, src3d, dst3d, w3d)


def kernel(x, edge_index, edge_weight, W1, b1, W2, b2):
    pad = E_PAD - E
    src3d = jnp.pad(edge_index[0], (0, pad)).reshape(NW, NCHUNK, CW)
    dst3d = jnp.pad(edge_index[1], (0, pad)).reshape(NW, NCHUNK, CW)
    w3d = jnp.pad(edge_weight, (0, pad)).reshape(NW, NCHUNK, CW)
    b1r = b1.reshape(1, D)
    b2r = b2.reshape(1, D)

    s1 = _tc_mm1(x, W1)
    p1 = _sc_aggregate(s1, src3d, dst3d, w3d)
    s2 = _tc_mm2(p1, b1r, W2)
    p2 = _sc_aggregate(s2, src3d, dst3d, w3d)
    return _tc_final(p2, b2r)
